# baseline (speedup 1.0000x reference)
import jax
import jax.numpy as jnp
from jax.experimental import pallas as pl
from jax.experimental.pallas import tpu as pltpu

_BS = 16
_ENC_LEN = 49
_D_IN = 1024
_D_MODEL = 512
_VOCAB = 100000
_STEPS = 20
_SOS = 1
_EOS = 2

_VBLK = 6272
_NBLK = 16
_LAST = _VOCAB - (_NBLK - 1) * _VBLK
_HCOLS = 32
_NEG_INF = float("-inf")


def _decode_body(enc_ref, wenc_ref, emb_hbm, w_hbm, tok_ref, lp_ref,
                 ctx_s, wb0, wb1, wlast, hbuf, vec_v, vec_s,
                 keyh, valh, lph, res_s, res_l, tokbuf, lpbuf,
                 sem_w, sem_l, sem_h, sem_p):
    lanes = jax.lax.broadcasted_iota(jnp.int32, (_BS, _HCOLS), 1)

    x = jnp.dot(enc_ref[...].reshape(_BS * _ENC_LEN, _D_IN), wenc_ref[...],
                preferred_element_type=jnp.float32)
    ctx_s[...] = jnp.mean(x.reshape(_BS, _ENC_LEN, _D_MODEL), axis=1)

    tokbuf[...] = jnp.where(lanes == 0, jnp.int32(_SOS), 0)
    lpbuf[...] = jnp.zeros_like(lpbuf)
    keyh[...] = jnp.full_like(keyh, -1)
    valh[...] = jnp.zeros_like(valh)
    lph[...] = jnp.zeros_like(lph)

    def _w_copy(i):
        if i < _NBLK - 1:
            return pltpu.make_async_copy(
                w_hbm.at[pl.ds(i * _VBLK, _VBLK), :],
                wb0 if i % 2 == 0 else wb1,
                sem_w.at[i % 2])
        return pltpu.make_async_copy(
            w_hbm.at[pl.ds(i * _VBLK, _LAST), :], wlast, sem_l)

    def _step(t, carry):
        prev, eosv, replay = carry
        eq = (keyh[...] == prev[:, None]) & (lanes < t)
        found = jnp.any(eq, axis=1)
        res_s[0, :] = jnp.min(jnp.where(eq, valh[...], jnp.int32(2**30)), axis=1)
        res_l[0, :] = jnp.min(jnp.where(eq, lph[...], jnp.inf), axis=1)

        @pl.when(replay == 0)
        def _stage():
            vec_v[0, :] = prev
            vec_v[1, :] = found.astype(jnp.int32)
            stage = pltpu.make_async_copy(vec_v, vec_s, sem_p)
            stage.start()
            stage.wait()

        n_found = vec_s[1, 0]
        for b in range(1, _BS):
            n_found = n_found + vec_s[1, b]
        need = n_found < _BS

        @pl.when(need)
        def _sweep():
            _w_copy(0).start()
            _w_copy(1).start()
            copies = [pltpu.make_async_copy(
                emb_hbm.at[pl.ds(vec_s[0, b], 1), :],
                hbuf.at[pl.ds(b, 1), :], sem_h) for b in range(_BS)]
            for c in copies:
                c.start()
            for c in copies:
                c.wait()
            h = hbuf[...] + ctx_s[...]
            m = jnp.full((_BS,), _NEG_INF, dtype=jnp.float32)
            s = jnp.zeros((_BS,), dtype=jnp.float32)
            a = jnp.zeros((_BS,), dtype=jnp.int32)
            for i in range(_NBLK):
                _w_copy(i).wait()
                if i == _NBLK - 1:
                    w = wlast[...]
                else:
                    w = (wb0 if i % 2 == 0 else wb1)[...]
                logits = jax.lax.dot_general(
                    h, w, (((1,), (1,)), ((), ())),
                    preferred_element_type=jnp.float32)
                col = jax.lax.broadcasted_iota(
                    jnp.int32, (_BS, w.shape[0]), 1) + i * _VBLK
                bm = jnp.max(logits, axis=1)
                cand = jnp.where(logits == bm[:, None], col, jnp.int32(2**30))
                barg = jnp.min(cand, axis=1)
                m_new = jnp.maximum(m, bm)
                s = (s * jnp.exp(m - m_new)
                     + jnp.sum(jnp.exp(logits - m_new[:, None]), axis=1))
                a = jnp.where(bm > m, barg, a)
                m = m_new
                if i + 2 < _NBLK:
                    _w_copy(i + 2).start()
            res_s[0, :] = a
            res_l[0, :] = -jnp.log(s)

        sampled = res_s[0, :]
        step_lp = res_l[0, :]
        keyh[...] = jnp.where(lanes == t, prev[:, None], keyh[...])
        valh[...] = jnp.where(lanes == t, sampled[:, None], valh[...])
        lph[...] = jnp.where(lanes == t, step_lp[:, None], lph[...])
        tokbuf[...] = jnp.where(lanes == t + 1, sampled[:, None], tokbuf[...])
        lpbuf[...] = jnp.where(lanes == t + 1, step_lp[:, None], lpbuf[...])
        eosv = jnp.minimum(
            eosv, jnp.where(sampled == _EOS, t + 1, _STEPS).astype(jnp.int32))
        replay = jnp.where(n_found == _BS, jnp.int32(1), replay)
        return sampled, eosv, replay

    prev0 = jnp.full((_BS,), _SOS, dtype=jnp.int32)
    eos0 = jnp.full((_BS,), _STEPS, dtype=jnp.int32)
    _, eosv, _ = jax.lax.fori_loop(
        0, _STEPS, _step, (prev0, eos0, jnp.int32(0)))

    tok_ref[...] = tokbuf[:, :_STEPS + 1]
    lp_masked = jnp.where(lanes > eosv[:, None], 0.0, lpbuf[...])
    lp_ref[...] = lp_masked[:, :_STEPS + 1]


def kernel(enc_x, W_enc, embed, W_out):
    toks, lp = pl.pallas_call(
        _decode_body,
        in_specs=[
            pl.BlockSpec(memory_space=pltpu.VMEM),
            pl.BlockSpec(memory_space=pltpu.VMEM),
            pl.BlockSpec(memory_space=pl.ANY),
            pl.BlockSpec(memory_space=pl.ANY),
        ],
        out_specs=[
            pl.BlockSpec(memory_space=pltpu.VMEM),
            pl.BlockSpec(memory_space=pltpu.VMEM),
        ],
        out_shape=[
            jax.ShapeDtypeStruct((_BS, _STEPS + 1), jnp.int32),
            jax.ShapeDtypeStruct((_BS, _STEPS + 1), jnp.float32),
        ],
        scratch_shapes=[
            pltpu.VMEM((_BS, _D_MODEL), jnp.float32),
            pltpu.VMEM((_VBLK, _D_MODEL), jnp.float32),
            pltpu.VMEM((_VBLK, _D_MODEL), jnp.float32),
            pltpu.VMEM((_LAST, _D_MODEL), jnp.float32),
            pltpu.VMEM((_BS, _D_MODEL), jnp.float32),
            pltpu.VMEM((2, _BS), jnp.int32),
            pltpu.SMEM((2, _BS), jnp.int32),
            pltpu.VMEM((_BS, _HCOLS), jnp.int32),
            pltpu.VMEM((_BS, _HCOLS), jnp.int32),
            pltpu.VMEM((_BS, _HCOLS), jnp.float32),
            pltpu.VMEM((1, _BS), jnp.int32),
            pltpu.VMEM((1, _BS), jnp.float32),
            pltpu.VMEM((_BS, _HCOLS), jnp.int32),
            pltpu.VMEM((_BS, _HCOLS), jnp.float32),
            pltpu.SemaphoreType.DMA((2,)),
            pltpu.SemaphoreType.DMA,
            pltpu.SemaphoreType.DMA,
            pltpu.SemaphoreType.DMA,
        ],
    )(enc_x, W_enc, embed, W_out.T)
    return toks, lp.reshape(_BS, 1, _STEPS + 1)


# speedup vs baseline: 1.0199x; 1.0199x over previous
import jax
import jax.numpy as jnp
from jax.experimental import pallas as pl
from jax.experimental.pallas import tpu as pltpu

_BS = 16
_ENC_LEN = 49
_D_IN = 1024
_D_MODEL = 512
_VOCAB = 100000
_STEPS = 20
_SOS = 1
_EOS = 2

_VBLK = 6272
_NBLK = 16
_LAST = _VOCAB - (_NBLK - 1) * _VBLK
_HCOLS = 32
_NEG_INF = float("-inf")


def _decode_body(enc_ref, wenc_ref, emb_hbm, w_hbm, tok_ref, lp_ref,
                 ctx_s, wb0, wb1, wlast, hbuf, vec_v, vec_s,
                 keyh, valh, lph, res_s, res_l, tokbuf, lpbuf,
                 sem_w, sem_l, sem_h, sem_p):
    lanes = jax.lax.broadcasted_iota(jnp.int32, (_BS, _HCOLS), 1)

    x = jnp.dot(enc_ref[...].reshape(_BS * _ENC_LEN, _D_IN), wenc_ref[...],
                preferred_element_type=jnp.float32)
    ctx_s[...] = jnp.mean(x.reshape(_BS, _ENC_LEN, _D_MODEL), axis=1)

    tokbuf[...] = jnp.where(lanes == 0, jnp.int32(_SOS), 0)
    lpbuf[...] = jnp.zeros_like(lpbuf)
    keyh[...] = jnp.full_like(keyh, -1)
    valh[...] = jnp.zeros_like(valh)
    lph[...] = jnp.zeros_like(lph)

    def _w_copy(i):
        if i < _NBLK - 1:
            return pltpu.make_async_copy(
                w_hbm.at[pl.ds(i * _VBLK, _VBLK), :],
                wb0 if i % 2 == 0 else wb1,
                sem_w.at[i % 2])
        return pltpu.make_async_copy(
            w_hbm.at[pl.ds(i * _VBLK, _LAST), :], wlast, sem_l)

    def _step(t, carry):
        prev, eosv, replay = carry
        eq = (keyh[...] == prev[:, None]) & (lanes < t)
        found = jnp.any(eq, axis=1)
        res_s[0, :] = jnp.min(jnp.where(eq, valh[...], jnp.int32(2**30)), axis=1)
        res_l[0, :] = jnp.min(jnp.where(eq, lph[...], jnp.inf), axis=1)

        @pl.when(replay == 0)
        def _stage():
            vec_v[0, :] = prev
            vec_v[1, :] = found.astype(jnp.int32)
            stage = pltpu.make_async_copy(vec_v, vec_s, sem_p)
            stage.start()
            stage.wait()

        n_found = vec_s[1, 0]
        for b in range(1, _BS):
            n_found = n_found + vec_s[1, b]
        need = n_found < _BS

        @pl.when(need)
        def _sweep():
            copies = [pltpu.make_async_copy(
                emb_hbm.at[pl.ds(vec_s[0, b], 1), :],
                hbuf.at[pl.ds(b, 1), :], sem_h) for b in range(_BS)]
            for c in copies:
                c.start()
            for c in copies:
                c.wait()
            h = hbuf[...] + ctx_s[...]

            _w_copy(0).start()
            _w_copy(1).start()
            m = jnp.full((_BS,), _NEG_INF, dtype=jnp.float32)
            s = jnp.zeros((_BS,), dtype=jnp.float32)
            a = jnp.zeros((_BS,), dtype=jnp.int32)
            for i in range(_NBLK):
                _w_copy(i).wait()
                if i == _NBLK - 1:
                    w = wlast[...]
                else:
                    w = (wb0 if i % 2 == 0 else wb1)[...]
                logits = jax.lax.dot_general(
                    h, w, (((1,), (1,)), ((), ())),
                    preferred_element_type=jnp.float32)
                col = jax.lax.broadcasted_iota(
                    jnp.int32, (_BS, w.shape[0]), 1) + i * _VBLK
                bm = jnp.max(logits, axis=1)
                cand = jnp.where(logits == bm[:, None], col, jnp.int32(2**30))
                barg = jnp.min(cand, axis=1)
                m_new = jnp.maximum(m, bm)
                s = (s * jnp.exp(m - m_new)
                     + jnp.sum(jnp.exp(logits - m_new[:, None]), axis=1))
                a = jnp.where(bm > m, barg, a)
                m = m_new
                if i + 2 < _NBLK:
                    _w_copy(i + 2).start()
            res_s[0, :] = a
            res_l[0, :] = -jnp.log(s)

        sampled = res_s[0, :]
        step_lp = res_l[0, :]
        keyh[...] = jnp.where(lanes == t, prev[:, None], keyh[...])
        valh[...] = jnp.where(lanes == t, sampled[:, None], valh[...])
        lph[...] = jnp.where(lanes == t, step_lp[:, None], lph[...])
        tokbuf[...] = jnp.where(lanes == t + 1, sampled[:, None], tokbuf[...])
        lpbuf[...] = jnp.where(lanes == t + 1, step_lp[:, None], lpbuf[...])
        eosv = jnp.minimum(
            eosv, jnp.where(sampled == _EOS, t + 1, _STEPS).astype(jnp.int32))
        replay = jnp.where(n_found == _BS, jnp.int32(1), replay)
        return sampled, eosv, replay

    prev0 = jnp.full((_BS,), _SOS, dtype=jnp.int32)
    eos0 = jnp.full((_BS,), _STEPS, dtype=jnp.int32)
    _, eosv, _ = jax.lax.fori_loop(
        0, _STEPS, _step, (prev0, eos0, jnp.int32(0)))

    tok_ref[...] = tokbuf[:, :_STEPS + 1]
    lp_masked = jnp.where(lanes > eosv[:, None], 0.0, lpbuf[...])
    lp_ref[...] = lp_masked[:, :_STEPS + 1]


def kernel(enc_x, W_enc, embed, W_out):
    toks, lp = pl.pallas_call(
        _decode_body,
        in_specs=[
            pl.BlockSpec(memory_space=pltpu.VMEM),
            pl.BlockSpec(memory_space=pltpu.VMEM),
            pl.BlockSpec(memory_space=pl.ANY),
            pl.BlockSpec(memory_space=pl.ANY),
        ],
        out_specs=[
            pl.BlockSpec(memory_space=pltpu.VMEM),
            pl.BlockSpec(memory_space=pltpu.VMEM),
        ],
        out_shape=[
            jax.ShapeDtypeStruct((_BS, _STEPS + 1), jnp.int32),
            jax.ShapeDtypeStruct((_BS, _STEPS + 1), jnp.float32),
        ],
        scratch_shapes=[
            pltpu.VMEM((_BS, _D_MODEL), jnp.float32),
            pltpu.VMEM((_VBLK, _D_MODEL), jnp.float32),
            pltpu.VMEM((_VBLK, _D_MODEL), jnp.float32),
            pltpu.VMEM((_LAST, _D_MODEL), jnp.float32),
            pltpu.VMEM((_BS, _D_MODEL), jnp.float32),
            pltpu.VMEM((2, _BS), jnp.int32),
            pltpu.SMEM((2, _BS), jnp.int32),
            pltpu.VMEM((_BS, _HCOLS), jnp.int32),
            pltpu.VMEM((_BS, _HCOLS), jnp.int32),
            pltpu.VMEM((_BS, _HCOLS), jnp.float32),
            pltpu.VMEM((1, _BS), jnp.int32),
            pltpu.VMEM((1, _BS), jnp.float32),
            pltpu.VMEM((_BS, _HCOLS), jnp.int32),
            pltpu.VMEM((_BS, _HCOLS), jnp.float32),
            pltpu.SemaphoreType.DMA((2,)),
            pltpu.SemaphoreType.DMA,
            pltpu.SemaphoreType.DMA,
            pltpu.SemaphoreType.DMA,
        ],
    )(enc_x, W_enc, embed, W_out.T)
    return toks, lp.reshape(_BS, 1, _STEPS + 1)
